# baseline (speedup 1.0000x reference)
import functools

import jax
import jax.numpy as jnp
from jax import lax
from jax.experimental import pallas as pl
from jax.experimental.pallas import tpu as pltpu
from jax.experimental.pallas import tpu_sc as plsc

BATCH = 4096
SEQ = 50
DIM = 128
LANES = 16
NCORE = 2
NSUB = 16
NWORKER = NCORE * NSUB
SENT_PER_W = BATCH // NWORKER
LGROUPS = DIM // LANES
SGROUPS = SENT_PER_W // LANES
NQ = 2
NQI = 2


def _embed_body(idx_hbm, table_hbm, out_hbm, blist_v, idxt_v, acc_v,
                iqsems, gsems):
    c = lax.axis_index("c")
    s = lax.axis_index("s")
    w = c * NSUB + s
    sent_base = pl.multiple_of(w * SENT_PER_W, SENT_PER_W)
    flat_base = sent_base * SEQ

    iota = lax.iota(jnp.int32, LANES)

    def ibody(g, carry):
        for blk in range(SGROUPS):
            vec = (iota + (blk * LANES)) * SEQ + (flat_base + g)
            blist_v[g, pl.ds(blk * LANES, LANES)] = vec
        pltpu.async_copy(idx_hbm.at[blist_v.at[g]], idxt_v.at[g],
                         iqsems.at[lax.rem(g, NQI)])
        return carry

    lax.fori_loop(0, SEQ, ibody, 0)

    zero = jnp.zeros((LANES,), jnp.float32)

    def zbody(r, carry):
        for l in range(LGROUPS):
            acc_v[r, pl.ds(l * LANES, LANES)] = zero
        return carry

    lax.fori_loop(0, SENT_PER_W, zbody, 0)

    def tbody(g, carry):
        pltpu.make_async_copy(idx_hbm.at[blist_v.at[g]], idxt_v.at[g],
                              iqsems.at[lax.rem(g, NQI)]).wait()
        pltpu.async_copy(table_hbm.at[idxt_v.at[g]], acc_v,
                         gsems.at[lax.rem(g, NQ)], add=True)
        return carry

    lax.fori_loop(0, SEQ, tbody, 0)

    def wbody(g, carry):
        pltpu.make_async_copy(table_hbm.at[idxt_v.at[g]], acc_v,
                              gsems.at[lax.rem(g, NQ)]).wait()
        return carry

    lax.fori_loop(0, SEQ, wbody, 0)

    scale = jnp.float32(1.0 / SEQ)

    def cbody(r, carry):
        for l in range(LGROUPS):
            sl = pl.ds(l * LANES, LANES)
            acc_v[r, sl] = acc_v[r, sl] * scale
        return carry

    lax.fori_loop(0, SENT_PER_W, cbody, 0)

    pltpu.sync_copy(acc_v, out_hbm.at[pl.ds(sent_base, SENT_PER_W)])


def _make():
    return functools.partial(
        pl.kernel,
        mesh=plsc.VectorSubcoreMesh(core_axis_name="c", subcore_axis_name="s"),
        out_type=jax.ShapeDtypeStruct((BATCH, DIM), jnp.float32),
        scratch_types=[
            pltpu.VMEM((SEQ, SENT_PER_W), jnp.int32),
            pltpu.VMEM((SEQ, SENT_PER_W), jnp.int32),
            pltpu.VMEM((SENT_PER_W, DIM), jnp.float32),
            pltpu.SemaphoreType.DMA((NQI,)),
            pltpu.SemaphoreType.DMA((NQ,)),
        ],
    )(_embed_body)


def kernel(indices, table):
    return _make()(indices.astype(jnp.int32).reshape(BATCH * SEQ), table)


# speedup vs baseline: 1.2252x; 1.2252x over previous
import functools

import jax
import jax.numpy as jnp
from jax import lax
from jax.experimental import pallas as pl
from jax.experimental.pallas import tpu as pltpu
from jax.experimental.pallas import tpu_sc as plsc

BATCH = 4096
SEQ = 50
DIM = 128
LANES = 16
NCORE = 2
NSUB = 16
NWORKER = NCORE * NSUB
SENT_PER_W = BATCH // NWORKER
LGROUPS = DIM // LANES
NQ = 2


def _embed_body(idx_hbm, table_hbm, out_hbm, idx_v, acc_v, isem, gsems):
    c = lax.axis_index("c")
    s = lax.axis_index("s")
    w = c * NSUB + s
    sent_base = pl.multiple_of(w * SENT_PER_W, SENT_PER_W)

    pltpu.async_copy(idx_hbm.at[w], idx_v, isem)

    zero = jnp.zeros((LANES,), jnp.float32)

    def zbody(r, carry):
        for l in range(LGROUPS):
            acc_v[r, pl.ds(l * LANES, LANES)] = zero
        return carry

    lax.fori_loop(0, SENT_PER_W, zbody, 0)
    pltpu.make_async_copy(idx_hbm.at[w], idx_v, isem).wait()

    def gbody(g, carry):
        pltpu.async_copy(table_hbm.at[idx_v.at[g]], acc_v,
                         gsems.at[lax.rem(g, NQ)], add=True)
        return carry

    lax.fori_loop(0, SEQ, gbody, 0)

    def wbody(g, carry):
        pltpu.make_async_copy(table_hbm.at[idx_v.at[g]], acc_v,
                              gsems.at[lax.rem(g, NQ)]).wait()
        return carry

    lax.fori_loop(0, SEQ, wbody, 0)

    scale = jnp.float32(1.0 / SEQ)

    def cbody(r, carry):
        for l in range(LGROUPS):
            sl = pl.ds(l * LANES, LANES)
            acc_v[r, sl] = acc_v[r, sl] * scale
        return carry

    lax.fori_loop(0, SENT_PER_W, cbody, 0)

    pltpu.sync_copy(acc_v, out_hbm.at[pl.ds(sent_base, SENT_PER_W)])


def _make():
    return functools.partial(
        pl.kernel,
        mesh=plsc.VectorSubcoreMesh(core_axis_name="c", subcore_axis_name="s"),
        out_type=jax.ShapeDtypeStruct((BATCH, DIM), jnp.float32),
        scratch_types=[
            pltpu.VMEM((SEQ, SENT_PER_W), jnp.int32),
            pltpu.VMEM((SENT_PER_W, DIM), jnp.float32),
            pltpu.SemaphoreType.DMA,
            pltpu.SemaphoreType.DMA((NQ,)),
        ],
    )(_embed_body)


def kernel(indices, table):
    idx = (indices.astype(jnp.int32)
           .reshape(NWORKER, SENT_PER_W, SEQ)
           .transpose(0, 2, 1))
    return _make()(idx, table)


# speedup vs baseline: 1.2295x; 1.0034x over previous
import functools

import jax
import jax.numpy as jnp
from jax import lax
from jax.experimental import pallas as pl
from jax.experimental.pallas import tpu as pltpu
from jax.experimental.pallas import tpu_sc as plsc

BATCH = 4096
SEQ = 50
DIM = 128
LANES = 16
NCORE = 2
NSUB = 16
NWORKER = NCORE * NSUB
SENT_PER_W = BATCH // NWORKER
LGROUPS = DIM // LANES
NQ = 2


def _embed_body(idx_hbm, table_hbm, out_hbm, idx_v, acc_v, isem, gsems):
    c = lax.axis_index("c")
    s = lax.axis_index("s")
    w = c * NSUB + s
    sent_base = pl.multiple_of(w * SENT_PER_W, SENT_PER_W)

    pltpu.async_copy(idx_hbm.at[:, pl.ds(sent_base, SENT_PER_W)], idx_v,
                     isem)

    zero = jnp.zeros((LANES,), jnp.float32)

    def zbody(r, carry):
        for l in range(LGROUPS):
            acc_v[r, pl.ds(l * LANES, LANES)] = zero
        return carry

    lax.fori_loop(0, SENT_PER_W, zbody, 0)
    pltpu.make_async_copy(idx_hbm.at[:, pl.ds(sent_base, SENT_PER_W)],
                          idx_v, isem).wait()

    def gbody(g, carry):
        pltpu.async_copy(table_hbm.at[idx_v.at[g]], acc_v,
                         gsems.at[lax.rem(g, NQ)], add=True)
        return carry

    lax.fori_loop(0, SEQ, gbody, 0)

    def wbody(g, carry):
        pltpu.make_async_copy(table_hbm.at[idx_v.at[g]], acc_v,
                              gsems.at[lax.rem(g, NQ)]).wait()
        return carry

    lax.fori_loop(0, SEQ, wbody, 0)

    scale = jnp.float32(1.0 / SEQ)

    def cbody(r, carry):
        for l in range(LGROUPS):
            sl = pl.ds(l * LANES, LANES)
            acc_v[r, sl] = acc_v[r, sl] * scale
        return carry

    lax.fori_loop(0, SENT_PER_W, cbody, 0)

    pltpu.sync_copy(acc_v, out_hbm.at[pl.ds(sent_base, SENT_PER_W)])


def _make():
    return functools.partial(
        pl.kernel,
        mesh=plsc.VectorSubcoreMesh(core_axis_name="c", subcore_axis_name="s"),
        out_type=jax.ShapeDtypeStruct((BATCH, DIM), jnp.float32),
        scratch_types=[
            pltpu.VMEM((SEQ, SENT_PER_W), jnp.int32),
            pltpu.VMEM((SENT_PER_W, DIM), jnp.float32),
            pltpu.SemaphoreType.DMA,
            pltpu.SemaphoreType.DMA((NQ,)),
        ],
    )(_embed_body)


def kernel(indices, table):
    return _make()(indices.astype(jnp.int32).T, table)
